# baseline (speedup 1.0000x reference)
import functools

import jax
import jax.numpy as jnp
from jax import lax
from jax.experimental import pallas as pl
from jax.experimental.pallas import tpu as pltpu
from jax.experimental.pallas import tpu_sc as plsc

VOCAB = 1000000
D = 32
B = 16384

G = 128 // D
NC = 2
NS = 16
NW = NC * NS
BPW = B // NW
CH = 128
NCH = BPW // CH
L = 16

_mesh = plsc.VectorSubcoreMesh(core_axis_name="c", subcore_axis_name="s")


@functools.partial(
    pl.kernel,
    mesh=_mesh,
    out_type=jax.ShapeDtypeStruct((B, D), jnp.float32),
    compiler_params=pltpu.CompilerParams(needs_layout_passes=False),
    scratch_types=[
        pltpu.VMEM((NCH, CH), jnp.int32),
        pltpu.VMEM((NCH, CH), jnp.int32),
        pltpu.VMEM((2, CH, 128), jnp.float32),
        pltpu.VMEM((CH, D), jnp.float32),
        pltpu.SemaphoreType.DMA,
    ],
)
def _sc_gather(idx_hbm, tbl_hbm, out_hbm, idx_v, g_v, grp_v, rows_v, sem):
    wid = lax.axis_index("s") * NC + lax.axis_index("c")
    base = wid * BPW
    for j in range(NCH):
        pltpu.sync_copy(idx_hbm.at[pl.ds(base + j * CH, CH)], idx_v.at[j])
    for j in range(NCH):
        for i in range(CH // L):
            s = pl.ds(i * L, L)
            g_v[j, s] = lax.shift_right_logical(idx_v[j, s], G.bit_length() - 1)

    lane = lax.iota(jnp.int32, L)
    cols = [jnp.full((L,), c, jnp.int32) for c in range(D)]

    def fire(j):
        return pltpu.async_copy(tbl_hbm.at[g_v.at[j]], grp_v.at[j % 2], sem)

    cp = fire(0)
    for j in range(NCH):
        nxt = fire(j + 1) if j + 1 < NCH else None
        cp.wait()
        gbuf = grp_v.at[j % 2]
        for b0 in range(0, CH, L):
            idx16 = idx_v[j, pl.ds(b0, L)]
            off = (idx16 & (G - 1)) * D
            bvec = lane + b0
            for c in range(D):
                v = plsc.load_gather(gbuf, [bvec, off + c])
                plsc.store_scatter(rows_v, [bvec, cols[c]], v)
        pltpu.sync_copy(rows_v, out_hbm.at[pl.ds(base + j * CH, CH)])
        cp = nxt


def kernel(down_ID, table):
    idx = down_ID.astype(jnp.int32)
    tbl = table.reshape(VOCAB // G, 128)
    return _sc_gather(idx, tbl)


# speedup vs baseline: 2.7795x; 2.7795x over previous
import functools

import jax
import jax.numpy as jnp
from jax import lax
from jax.experimental import pallas as pl
from jax.experimental.pallas import tpu as pltpu
from jax.experimental.pallas import tpu_sc as plsc

VOCAB = 1000000
D = 32
B = 16384

G = 8
NC = 2
NS = 16
NW = NC * NS
BPW = B // NW

_mesh = plsc.VectorSubcoreMesh(core_axis_name="c", subcore_axis_name="s")


@functools.partial(
    pl.kernel,
    mesh=_mesh,
    out_type=jax.ShapeDtypeStruct((B, D), jnp.float32),
    compiler_params=pltpu.CompilerParams(needs_layout_passes=False),
    scratch_types=[
        pltpu.VMEM((BPW,), jnp.int32),
        pltpu.VMEM((BPW, D), jnp.float32),
        pltpu.SemaphoreType.DMA,
    ],
)
def _sc_gather(idx_hbm, tbl_hbm, out_hbm, idx_v, rows_v, sem):
    wid = lax.axis_index("s") * NC + lax.axis_index("c")
    base = wid * BPW
    pltpu.sync_copy(idx_hbm.at[pl.ds(base, BPW)], idx_v)

    for b0 in range(0, BPW, 16):
        v = idx_v[pl.ds(b0, 16)]
        for l in range(16):
            idx = v[l]
            pltpu.async_copy(
                tbl_hbm.at[idx >> 3, idx & 7], rows_v.at[b0 + l], sem
            )
    pltpu.make_async_copy(
        out_hbm.at[pl.ds(base, BPW)], rows_v, sem
    ).wait()
    pltpu.sync_copy(rows_v, out_hbm.at[pl.ds(base, BPW)])


def kernel(down_ID, table):
    idx = down_ID.astype(jnp.int32)
    tbl = table.reshape(VOCAB // G, G, D)
    return _sc_gather(idx, tbl)
